# Initial kernel scaffold; baseline (speedup 1.0000x reference)
import jax
import jax.numpy as jnp
from jax.experimental import pallas as pl


def kernel(x):
    raise NotImplementedError("write your pallas kernel here")


# speedup vs baseline: 1.5541x; 1.5541x over previous
import jax
import jax.numpy as jnp
from jax.experimental import pallas as pl

_N = 512
_T = 64
_NA = 1


def _body(x_ref, d1_ref, d2_ref):
    t = pl.program_id(0)
    xv = x_ref[...]
    valid = jnp.where(jnp.isnan(xv), 0.0, 1.0)
    lane = jax.lax.broadcasted_iota(jnp.int32, (_N, _T), 1)
    vcol = jnp.sum(jnp.where(lane == t, valid, 0.0), axis=1, keepdims=True)
    row = jax.lax.broadcasted_iota(jnp.int32, (_N, _N - 1), 0)
    col = jax.lax.broadcasted_iota(jnp.int32, (_N, _N - 1), 1)
    d1_ref[...] = jnp.where(row == col + 1, vcol, 0.0)
    col2 = jax.lax.broadcasted_iota(jnp.int32, (_N, _T - _NA - 1), 1)
    d2_ref[...] = jnp.where(col2 == t - (_NA + 1), vcol, 0.0)


def kernel(x):
    xt = jnp.transpose(x[0])
    d1, d2 = pl.pallas_call(
        _body,
        grid=(_T,),
        in_specs=[pl.BlockSpec((_N, _T), lambda t: (0, 0))],
        out_specs=[
            pl.BlockSpec((_N, _N - 1), lambda t: (t, 0)),
            pl.BlockSpec((_N, _T - _NA - 1), lambda t: (t, 0)),
        ],
        out_shape=[
            jax.ShapeDtypeStruct((_T * _N, _N - 1), jnp.float32),
            jax.ShapeDtypeStruct((_T * _N, _T - _NA - 1), jnp.float32),
        ],
    )(xt)
    return d1[None], d2[None]
